# Initial kernel scaffold; baseline (speedup 1.0000x reference)
import jax
import jax.numpy as jnp
from jax.experimental import pallas as pl


def kernel(expression, bin_embedding, continuous_projection):
    raise NotImplementedError("write your pallas kernel here")


# speedup vs baseline: 1.1521x; 1.1521x over previous
import jax
import jax.numpy as jnp
from jax import lax
from jax.experimental import pallas as pl
from jax.experimental.pallas import tpu as pltpu
from jax.experimental.pallas import tpu_sc as plsc

EMBED_DIM = 64
NUM_BINS = 50
ALPHA = 0.1
BATCH = 1024
G = 1000

NC = 2
NS = 16
LANES = 16
NW = NC * NS
TOTAL = BATCH * G
CHUNK = TOTAL // NW
BLOCK = 800
SUBS = BLOCK // LANES
NBLK = CHUNK // BLOCK


def _prep_body(x_ref, idx_ref, cn_ref):
    x = x_ref[...]
    n = jnp.log(x + 1.0)
    mn = jnp.min(n)
    mx = jnp.max(n)
    r = (n - mn) / (mx - mn + 1e-8)
    idx = jnp.floor(r * (NUM_BINS - 1)).astype(jnp.int32)
    idx_ref[...] = jnp.clip(idx, 0, NUM_BINS - 1) * EMBED_DIM
    cn_ref[...] = ALPHA * n


def _sc_body(idx_hbm, cn_hbm, tab_hbm, proj_hbm, out_hbm,
             tab_v, proj_v, idx_v, cn_v, outb0, outb1, sem0, sem1):
    wid = lax.axis_index("s") * NC + lax.axis_index("c")
    base = wid * CHUNK
    pltpu.sync_copy(tab_hbm, tab_v)
    pltpu.sync_copy(proj_hbm, proj_v)
    sems = (sem0, sem1)
    bufs = (outb0, outb1)
    iota64 = lax.iota(jnp.int32, LANES) * EMBED_DIM
    obase = base * EMBED_DIM

    def do_block(blk, b):
        @pl.when(blk >= 2)
        def _():
            pltpu.make_async_copy(
                bufs[b], out_hbm.at[pl.ds(obase, BLOCK * EMBED_DIM)],
                sems[b]
            ).wait()
        bstart = base + blk * BLOCK
        pltpu.sync_copy(idx_hbm.at[pl.ds(bstart, BLOCK)], idx_v)
        pltpu.sync_copy(cn_hbm.at[pl.ds(bstart, BLOCK)], cn_v)

        def sub(s, _):
            idxv = idx_v[pl.ds(s * LANES, LANES)]
            sv = cn_v[pl.ds(s * LANES, LANES)]
            elemv = iota64 + s * (LANES * EMBED_DIM)
            for j in range(EMBED_DIM):
                jc = jnp.full((LANES,), j, jnp.int32)
                col = plsc.load_gather(tab_v, [idxv + jc])
                pj = plsc.load_gather(proj_v, [jc])
                plsc.store_scatter(bufs[b], [elemv + jc], col + sv * pj)
            return 0

        lax.fori_loop(0, SUBS, sub, 0)
        pltpu.async_copy(
            bufs[b],
            out_hbm.at[pl.ds(bstart * EMBED_DIM, BLOCK * EMBED_DIM)],
            sems[b])

    def pair(i, _):
        do_block(i * 2, 0)
        do_block(i * 2 + 1, 1)
        return 0

    lax.fori_loop(0, NBLK // 2, pair, 0)
    for b in range(2):
        pltpu.make_async_copy(
            bufs[b], out_hbm.at[pl.ds(obase, BLOCK * EMBED_DIM)], sems[b]
        ).wait()


_prep = pl.pallas_call(
    _prep_body,
    out_shape=(
        jax.ShapeDtypeStruct((TOTAL // 128, 128), jnp.int32),
        jax.ShapeDtypeStruct((TOTAL // 128, 128), jnp.float32),
    ),
    in_specs=[pl.BlockSpec(memory_space=pltpu.VMEM)],
    out_specs=(pl.BlockSpec(memory_space=pltpu.VMEM),
               pl.BlockSpec(memory_space=pltpu.VMEM)),
)

_main = pl.kernel(
    _sc_body,
    out_type=jax.ShapeDtypeStruct((TOTAL * EMBED_DIM,), jnp.float32),
    mesh=plsc.VectorSubcoreMesh(core_axis_name="c", subcore_axis_name="s",
                                num_cores=NC, num_subcores=NS),
    compiler_params=pltpu.CompilerParams(needs_layout_passes=False),
    scratch_types=[
        pltpu.VMEM((NUM_BINS * EMBED_DIM,), jnp.float32),
        pltpu.VMEM((EMBED_DIM,), jnp.float32),
        pltpu.VMEM((BLOCK,), jnp.int32),
        pltpu.VMEM((BLOCK,), jnp.float32),
        pltpu.VMEM((BLOCK * EMBED_DIM,), jnp.float32),
        pltpu.VMEM((BLOCK * EMBED_DIM,), jnp.float32),
        pltpu.SemaphoreType.DMA,
        pltpu.SemaphoreType.DMA,
    ],
)


def kernel(expression, bin_embedding, continuous_projection):
    x = expression.reshape(TOTAL // 128, 128)
    idx, cn = _prep(x)
    out = _main(idx.reshape(TOTAL), cn.reshape(TOTAL),
                bin_embedding.reshape(NUM_BINS * EMBED_DIM),
                continuous_projection)
    return out.reshape(BATCH, G, EMBED_DIM)


# speedup vs baseline: 1.3272x; 1.1519x over previous
import jax
import jax.numpy as jnp
from jax import lax
from jax.experimental import pallas as pl
from jax.experimental.pallas import tpu as pltpu
from jax.experimental.pallas import tpu_sc as plsc

EMBED_DIM = 64
NUM_BINS = 50
ALPHA = 0.1
BATCH = 1024
G = 1000

NC = 2
NS = 16
LANES = 16
NW = NC * NS
TOTAL = BATCH * G
CHUNK = TOTAL // NW
BLOCK = 640
SUBS = BLOCK // LANES
NBLK = CHUNK // BLOCK


def _prep_body(x_ref, idx_ref, cn_ref):
    x = x_ref[...]
    n = jnp.log(x + 1.0)
    mn = jnp.min(n)
    mx = jnp.max(n)
    r = (n - mn) / (mx - mn + 1e-8)
    idx = jnp.floor(r * (NUM_BINS - 1)).astype(jnp.int32)
    idx_ref[...] = jnp.clip(idx, 0, NUM_BINS - 1) * EMBED_DIM
    cn_ref[...] = ALPHA * n


def _sc_body(idx_hbm, cn_hbm, tab_hbm, proj_hbm, out_hbm,
             tab_v, proj_v, idx_v, cn_v, outb0, outb1, sem0, sem1):
    wid = lax.axis_index("s") * NC + lax.axis_index("c")
    base = wid * CHUNK
    pltpu.sync_copy(tab_hbm, tab_v)
    pltpu.sync_copy(proj_hbm, proj_v)
    sems = (sem0, sem1)
    bufs = (outb0, outb1)
    iota = lax.iota(jnp.int32, LANES)

    def do_block(blk, b):
        @pl.when(blk >= 2)
        def _():
            pltpu.make_async_copy(
                bufs[b], out_hbm.at[pl.ds(base, BLOCK), :], sems[b]
            ).wait()
        bstart = base + blk * BLOCK
        pltpu.sync_copy(idx_hbm.at[pl.ds(bstart, BLOCK)], idx_v)
        pltpu.sync_copy(cn_hbm.at[pl.ds(bstart, BLOCK)], cn_v)

        @plsc.parallel_loop(0, SUBS, 1, unroll=4)
        def _(s):
            idxv = idx_v[pl.ds(s * LANES, LANES)]
            sv = cn_v[pl.ds(s * LANES, LANES)]
            elemv = iota + s * LANES
            for j in range(EMBED_DIM):
                jc = jnp.full((LANES,), j, jnp.int32)
                col = plsc.load_gather(tab_v, [idxv + jc])
                pj = plsc.load_gather(proj_v, [jc])
                plsc.store_scatter(bufs[b], [elemv, jc], col + sv * pj)

        pltpu.async_copy(bufs[b], out_hbm.at[pl.ds(bstart, BLOCK), :],
                         sems[b])

    def pair(i, _):
        do_block(i * 2, 0)
        do_block(i * 2 + 1, 1)
        return 0

    lax.fori_loop(0, NBLK // 2, pair, 0)
    for b in range(2):
        pltpu.make_async_copy(
            bufs[b], out_hbm.at[pl.ds(base, BLOCK), :], sems[b]
        ).wait()


_prep = pl.pallas_call(
    _prep_body,
    out_shape=(
        jax.ShapeDtypeStruct((TOTAL,), jnp.int32),
        jax.ShapeDtypeStruct((TOTAL,), jnp.float32),
    ),
    in_specs=[pl.BlockSpec(memory_space=pltpu.VMEM)],
    out_specs=(pl.BlockSpec(memory_space=pltpu.VMEM),
               pl.BlockSpec(memory_space=pltpu.VMEM)),
)

_main = pl.kernel(
    _sc_body,
    out_type=jax.ShapeDtypeStruct((TOTAL, EMBED_DIM), jnp.float32),
    mesh=plsc.VectorSubcoreMesh(core_axis_name="c", subcore_axis_name="s",
                                num_cores=NC, num_subcores=NS),
    compiler_params=pltpu.CompilerParams(needs_layout_passes=False,
                                         use_tc_tiling_on_sc=False),
    scratch_types=[
        pltpu.VMEM((NUM_BINS * EMBED_DIM,), jnp.float32),
        pltpu.VMEM((EMBED_DIM,), jnp.float32),
        pltpu.VMEM((BLOCK,), jnp.int32),
        pltpu.VMEM((BLOCK,), jnp.float32),
        pltpu.VMEM((BLOCK, EMBED_DIM), jnp.float32),
        pltpu.VMEM((BLOCK, EMBED_DIM), jnp.float32),
        pltpu.SemaphoreType.DMA,
        pltpu.SemaphoreType.DMA,
    ],
)


def kernel(expression, bin_embedding, continuous_projection):
    x = expression.reshape(TOTAL)
    idx, cn = _prep(x)
    out = _main(idx, cn,
                bin_embedding.reshape(NUM_BINS * EMBED_DIM),
                continuous_projection)
    return out.reshape(BATCH, G, EMBED_DIM)


# speedup vs baseline: 1.3282x; 1.0007x over previous
import jax
import jax.numpy as jnp
from jax import lax
from jax.experimental import pallas as pl
from jax.experimental.pallas import tpu as pltpu
from jax.experimental.pallas import tpu_sc as plsc

EMBED_DIM = 64
NUM_BINS = 50
ALPHA = 0.1
BATCH = 1024
G = 1000

NC = 2
NS = 16
LANES = 16
NW = NC * NS
TOTAL = BATCH * G
CHUNK = TOTAL // NW
BLOCK = 640
SUBS = BLOCK // LANES
NBLK = CHUNK // BLOCK


def _prep_body(n_ref, idx_ref, cn_ref):
    n = n_ref[...]
    mn = jnp.min(n)
    mx = jnp.max(n)
    r = (n - mn) * (1.0 / (mx - mn + 1e-8))
    idx = jnp.floor(r * (NUM_BINS - 1)).astype(jnp.int32)
    idx_ref[...] = jnp.clip(idx, 0, NUM_BINS - 1) * EMBED_DIM
    cn_ref[...] = ALPHA * n


def _sc_body(idx_hbm, cn_hbm, tab_hbm, proj_hbm, out_hbm,
             tab_v, proj_v, idx_v, cn_v, outb0, outb1, sem0, sem1):
    wid = lax.axis_index("s") * NC + lax.axis_index("c")
    base = wid * CHUNK
    pltpu.sync_copy(tab_hbm, tab_v)
    pltpu.sync_copy(proj_hbm, proj_v)
    sems = (sem0, sem1)
    bufs = (outb0, outb1)
    iota = lax.iota(jnp.int32, LANES)

    def do_block(blk, b):
        @pl.when(blk >= 2)
        def _():
            pltpu.make_async_copy(
                bufs[b], out_hbm.at[pl.ds(base, BLOCK), :], sems[b]
            ).wait()
        bstart = base + blk * BLOCK
        pltpu.sync_copy(idx_hbm.at[pl.ds(bstart, BLOCK)], idx_v)
        pltpu.sync_copy(cn_hbm.at[pl.ds(bstart, BLOCK)], cn_v)

        @plsc.parallel_loop(0, SUBS, 1, unroll=4)
        def _(s):
            idxv = idx_v[pl.ds(s * LANES, LANES)]
            sv = cn_v[pl.ds(s * LANES, LANES)]
            elemv = iota + s * LANES
            for j in range(EMBED_DIM):
                jc = jnp.full((LANES,), j, jnp.int32)
                col = plsc.load_gather(tab_v, [idxv + jc])
                pj = proj_v[pl.ds(j * LANES, LANES)]
                plsc.store_scatter(bufs[b], [elemv, jc], col + sv * pj)

        pltpu.async_copy(bufs[b], out_hbm.at[pl.ds(bstart, BLOCK), :],
                         sems[b])

    def pair(i, _):
        do_block(i * 2, 0)
        do_block(i * 2 + 1, 1)
        return 0

    lax.fori_loop(0, NBLK // 2, pair, 0)
    for b in range(2):
        pltpu.make_async_copy(
            bufs[b], out_hbm.at[pl.ds(base, BLOCK), :], sems[b]
        ).wait()


_prep = pl.pallas_call(
    _prep_body,
    out_shape=(
        jax.ShapeDtypeStruct((TOTAL,), jnp.int32),
        jax.ShapeDtypeStruct((TOTAL,), jnp.float32),
    ),
    in_specs=[pl.BlockSpec(memory_space=pltpu.VMEM)],
    out_specs=(pl.BlockSpec(memory_space=pltpu.VMEM),
               pl.BlockSpec(memory_space=pltpu.VMEM)),
)

_main = pl.kernel(
    _sc_body,
    out_type=jax.ShapeDtypeStruct((TOTAL, EMBED_DIM), jnp.float32),
    mesh=plsc.VectorSubcoreMesh(core_axis_name="c", subcore_axis_name="s",
                                num_cores=NC, num_subcores=NS),
    compiler_params=pltpu.CompilerParams(needs_layout_passes=False,
                                         use_tc_tiling_on_sc=False),
    scratch_types=[
        pltpu.VMEM((NUM_BINS * EMBED_DIM,), jnp.float32),
        pltpu.VMEM((EMBED_DIM * LANES,), jnp.float32),
        pltpu.VMEM((BLOCK,), jnp.int32),
        pltpu.VMEM((BLOCK,), jnp.float32),
        pltpu.VMEM((BLOCK, EMBED_DIM), jnp.float32),
        pltpu.VMEM((BLOCK, EMBED_DIM), jnp.float32),
        pltpu.SemaphoreType.DMA,
        pltpu.SemaphoreType.DMA,
    ],
)


def kernel(expression, bin_embedding, continuous_projection):
    n = jnp.log(expression.reshape(TOTAL) + 1.0)
    idx, cn = _prep(n)
    out = _main(idx, cn,
                bin_embedding.reshape(NUM_BINS * EMBED_DIM),
                jnp.repeat(continuous_projection, LANES))
    return out.reshape(BATCH, G, EMBED_DIM)


# speedup vs baseline: 3.4132x; 2.5699x over previous
import jax
import jax.numpy as jnp
from jax import lax
from jax.experimental import pallas as pl
from jax.experimental.pallas import tpu as pltpu
from jax.experimental.pallas import tpu_sc as plsc

EMBED_DIM = 64
NUM_BINS = 50
ALPHA = 0.1
BATCH = 1024
G = 1000

NC = 2
NS = 16
LANES = 16
NW = NC * NS
TOTAL = BATCH * G
QPG = 4
NQ = G * QPG
QPW = NQ // NW
QJ = EMBED_DIM // QPG
QF = QJ * BATCH
MGRP = BATCH // LANES


def _prep_body(n_ref, idx_ref, cn_ref):
    n = n_ref[...]
    mn = jnp.min(n)
    mx = jnp.max(n)
    r = (n - mn) / (mx - mn + 1e-8)
    idx = jnp.floor(r * (NUM_BINS - 1)).astype(jnp.int32)
    idx_ref[...] = jnp.clip(idx, 0, NUM_BINS - 1) * EMBED_DIM
    cn_ref[...] = ALPHA * n


def _sc_body(idx_hbm, cn_hbm, tab_hbm, proj_hbm, out_hbm,
             tab_v, proj_v, idx_v, cn_v, outb0, outb1, sem0, sem1):
    wid = lax.axis_index("s") * NC + lax.axis_index("c")
    q0 = wid * QPW
    pltpu.sync_copy(tab_hbm, tab_v)
    pltpu.sync_copy(proj_hbm, proj_v)
    sems = (sem0, sem1)
    bufs = (outb0, outb1)

    def do_block(i, b):
        q = q0 + i
        @pl.when(i >= 2)
        def _():
            pltpu.make_async_copy(
                bufs[b], out_hbm.at[pl.ds(0, QF)], sems[b]
            ).wait()
        g = q // QPG
        jbase = (q % QPG) * QJ
        pltpu.sync_copy(idx_hbm.at[pl.ds(g * BATCH, BATCH)], idx_v)
        pltpu.sync_copy(cn_hbm.at[pl.ds(g * BATCH, BATCH)], cn_v)

        @plsc.parallel_loop(0, MGRP, 1, unroll=2)
        def _(m):
            idxv = idx_v[pl.ds(m * LANES, LANES)] + jbase
            sv = cn_v[pl.ds(m * LANES, LANES)]
            mbase = (m // 8) * 1024 + (m % 8) * LANES
            for jj in range(QJ):
                col = plsc.load_gather(tab_v, [idxv + jj])
                pj = proj_v[pl.ds((jbase + jj) * LANES, LANES)]
                loc = (jj // 8) * 8192 + (jj % 8) * 128
                bufs[b][pl.ds(mbase + loc, LANES)] = col + sv * pj

        pltpu.async_copy(bufs[b], out_hbm.at[pl.ds(q * QF, QF)], sems[b])

    def pair(p, _):
        do_block(p * 2, 0)
        do_block(p * 2 + 1, 1)
        return 0

    lax.fori_loop(0, (QPW - 1) // 2, pair, 0)
    do_block(QPW - 1, 0)
    for b in range(2):
        pltpu.make_async_copy(
            bufs[b], out_hbm.at[pl.ds(0, QF)], sems[b]
        ).wait()


_prep = pl.pallas_call(
    _prep_body,
    out_shape=(
        jax.ShapeDtypeStruct((TOTAL,), jnp.int32),
        jax.ShapeDtypeStruct((TOTAL,), jnp.float32),
    ),
    in_specs=[pl.BlockSpec(memory_space=pltpu.VMEM)],
    out_specs=(pl.BlockSpec(memory_space=pltpu.VMEM),
               pl.BlockSpec(memory_space=pltpu.VMEM)),
)

_main = pl.kernel(
    _sc_body,
    out_type=jax.ShapeDtypeStruct((TOTAL * EMBED_DIM,), jnp.float32),
    mesh=plsc.VectorSubcoreMesh(core_axis_name="c", subcore_axis_name="s",
                                num_cores=NC, num_subcores=NS),
    compiler_params=pltpu.CompilerParams(needs_layout_passes=False,
                                         use_tc_tiling_on_sc=False),
    scratch_types=[
        pltpu.VMEM((NUM_BINS * EMBED_DIM,), jnp.float32),
        pltpu.VMEM((EMBED_DIM * LANES,), jnp.float32),
        pltpu.VMEM((BATCH,), jnp.int32),
        pltpu.VMEM((BATCH,), jnp.float32),
        pltpu.VMEM((QF,), jnp.float32),
        pltpu.VMEM((QF,), jnp.float32),
        pltpu.SemaphoreType.DMA,
        pltpu.SemaphoreType.DMA,
    ],
)


def kernel(expression, bin_embedding, continuous_projection):
    n = jnp.log(expression.T.reshape(TOTAL) + 1.0)
    idx, cn = _prep(n)
    out = _main(idx, cn,
                bin_embedding.reshape(NUM_BINS * EMBED_DIM),
                jnp.repeat(continuous_projection, LANES))
    p = out.reshape(G, 8, 8, 8, 128)
    return p.transpose(2, 4, 0, 1, 3).reshape(BATCH, G, EMBED_DIM)
